# baseline (speedup 1.0000x reference)
import functools

import jax
import jax.numpy as jnp
from jax import lax
from jax.experimental import pallas as pl
from jax.experimental.pallas import tpu as pltpu
from jax.experimental.pallas import tpu_sc as plsc

_NUM_EXPERTS = 64
_BLOCK_M = 2048
_N_TOKENS = 16384
_N_CHUNKS = 4
_CHUNK = _N_TOKENS // _N_CHUNKS
_N_WORKERS = 32
_TOK_PER_W = _CHUNK // _N_WORKERS
_GROUPS = _TOK_PER_W // 16


def _matmul_body(x_ref, w_ref, o_ref):
    o_ref[...] = jax.lax.dot_general(
        x_ref[...], w_ref[...], (((1,), (1,)), ((), ())),
        preferred_element_type=jnp.float32,
    )


def _logits_tc(x, W_g):
    n_tokens, dim = x.shape
    return pl.pallas_call(
        _matmul_body,
        grid=(n_tokens // _BLOCK_M,),
        in_specs=[
            pl.BlockSpec((_BLOCK_M, dim), lambda i: (i, 0)),
            pl.BlockSpec((_NUM_EXPERTS, dim), lambda i: (0, 0)),
        ],
        out_specs=pl.BlockSpec((_BLOCK_M, _NUM_EXPERTS), lambda i: (i, 0)),
        out_shape=jax.ShapeDtypeStruct((n_tokens, _NUM_EXPERTS), jnp.float32),
        compiler_params=pltpu.CompilerParams(
            dimension_semantics=("arbitrary",),
            vmem_limit_bytes=64 * 1024 * 1024,
        ),
    )(x, W_g)


def _sc_topk_body(logits_hbm, out_hbm, logits_v, out_v):
    wid = lax.axis_index("s") * 2 + lax.axis_index("c")
    base = wid * _TOK_PER_W
    pltpu.sync_copy(logits_hbm.at[pl.ds(base, _TOK_PER_W)], logits_v)
    lane = lax.broadcasted_iota(jnp.int32, (16,), 0)
    zeros = jnp.zeros((16,), jnp.int32)
    neg = jnp.full((16,), -jnp.inf, jnp.float32)

    def group_body(g, carry):
        row = g * 16 + lane
        v1 = neg
        v2 = neg
        for e in range(_NUM_EXPERTS):
            x = plsc.load_gather(logits_v, [row, zeros + e])
            v1, v2 = jnp.maximum(v1, x), jnp.maximum(v2, jnp.minimum(v1, x))
        e2 = jnp.exp(v2 - v1)
        denom = 1.0 + e2
        plsc.store_scatter(out_v, [row, zeros], 1.0 / denom)
        plsc.store_scatter(out_v, [row, zeros + 1], e2 / denom)
        return carry

    lax.fori_loop(0, _GROUPS, group_body, 0)
    pltpu.sync_copy(out_v, out_hbm.at[pl.ds(base, _TOK_PER_W)])


_sc_topk = pl.kernel(
    _sc_topk_body,
    out_type=jax.ShapeDtypeStruct((_CHUNK, 2), jnp.float32),
    mesh=plsc.VectorSubcoreMesh(core_axis_name="c", subcore_axis_name="s"),
    compiler_params=pltpu.CompilerParams(needs_layout_passes=False),
    scratch_types=[
        pltpu.VMEM((_TOK_PER_W, _NUM_EXPERTS), jnp.float32),
        pltpu.VMEM((_TOK_PER_W, 2), jnp.float32),
    ],
)


@jax.jit
def kernel(x, W_g):
    outs = []
    for c in range(_N_CHUNKS):
        logits = _logits_tc(lax.slice_in_dim(x, c * _CHUNK, (c + 1) * _CHUNK), W_g)
        outs.append(_sc_topk(logits))
    return jnp.concatenate(outs, axis=0)


# speedup vs baseline: 3.2510x; 3.2510x over previous
import functools

import jax
import jax.numpy as jnp
from jax.experimental import pallas as pl
from jax.experimental.pallas import tpu as pltpu

_NUM_EXPERTS = 64
_BLOCK_M = 512
_N_STREAMS = 4


def _top2_softmax(logits):
    v1 = jnp.max(logits, axis=-1, keepdims=True)
    iota = jax.lax.broadcasted_iota(jnp.int32, logits.shape, 1)
    idx1 = jnp.min(
        jnp.where(logits == v1, iota, _NUM_EXPERTS), axis=-1, keepdims=True
    )
    v2 = jnp.max(jnp.where(iota == idx1, -jnp.inf, logits), axis=-1, keepdims=True)
    e2 = jnp.exp(v2 - v1)
    denom = 1.0 + e2
    return jnp.concatenate([1.0 / denom, e2 / denom], axis=-1)


def _gating_body(*refs):
    x_refs = refs[:_N_STREAMS]
    w_ref = refs[_N_STREAMS]
    o_refs = refs[_N_STREAMS + 1:]
    w = w_ref[...]
    dims = (((1,), (1,)), ((), ()))
    for x_ref, o_ref in zip(x_refs, o_refs):
        logits = jax.lax.dot_general(
            x_ref[...], w, dims, preferred_element_type=jnp.float32
        )
        o_ref[...] = _top2_softmax(logits)


@functools.partial(jax.jit, static_argnames=("interpret",))
def kernel(x, W_g, interpret=False):
    n_tokens, dim = x.shape
    per_stream = n_tokens // _N_STREAMS
    blocks = per_stream // _BLOCK_M
    outs = pl.pallas_call(
        _gating_body,
        grid=(blocks,),
        in_specs=[
            pl.BlockSpec((_BLOCK_M, dim), functools.partial(
                lambda s, i: (i + s * blocks, 0), s))
            for s in range(_N_STREAMS)
        ] + [pl.BlockSpec((_NUM_EXPERTS, dim), lambda i: (0, 0))],
        out_specs=[
            pl.BlockSpec((_BLOCK_M, 2), lambda i: (i, 0))
            for _ in range(_N_STREAMS)
        ],
        out_shape=[
            jax.ShapeDtypeStruct((per_stream, 2), jnp.float32)
            for _ in range(_N_STREAMS)
        ],
        compiler_params=pltpu.CompilerParams(
            dimension_semantics=("arbitrary",),
            vmem_limit_bytes=64 * 1024 * 1024,
        ),
        interpret=interpret,
    )(*([x] * _N_STREAMS), W_g)
    return jnp.concatenate(outs, axis=0)


# speedup vs baseline: 3.3043x; 1.0164x over previous
import functools

import jax
import jax.numpy as jnp
from jax.experimental import pallas as pl
from jax.experimental.pallas import tpu as pltpu

_NUM_EXPERTS = 64
_BLOCK_M = 1024


def _top2_softmax(logits):
    v1 = jnp.max(logits, axis=-1, keepdims=True)
    iota = jax.lax.broadcasted_iota(jnp.int32, logits.shape, 1)
    idx1 = jnp.min(
        jnp.where(logits == v1, iota, _NUM_EXPERTS), axis=-1, keepdims=True
    )
    v2 = jnp.max(jnp.where(iota == idx1, -jnp.inf, logits), axis=-1, keepdims=True)
    e2 = jnp.exp(v2 - v1)
    denom = 1.0 + e2
    return jnp.concatenate([1.0 / denom, e2 / denom], axis=-1)


def _gating_body(xa_ref, xb_ref, w_ref, oa_ref, ob_ref):
    w = w_ref[...].astype(jnp.bfloat16)
    dims = (((1,), (1,)), ((), ()))
    la = jax.lax.dot_general(
        xa_ref[...].astype(jnp.bfloat16), w, dims,
        preferred_element_type=jnp.float32,
    )
    oa_ref[...] = _top2_softmax(la)
    lb = jax.lax.dot_general(
        xb_ref[...].astype(jnp.bfloat16), w, dims,
        preferred_element_type=jnp.float32,
    )
    ob_ref[...] = _top2_softmax(lb)


@functools.partial(jax.jit, static_argnames=("interpret",))
def kernel(x, W_g, interpret=False):
    n_tokens, dim = x.shape
    half_blocks = n_tokens // (2 * _BLOCK_M)
    grid = (half_blocks,)
    out_a, out_b = pl.pallas_call(
        _gating_body,
        grid=grid,
        in_specs=[
            pl.BlockSpec((_BLOCK_M, dim), lambda i: (i, 0)),
            pl.BlockSpec((_BLOCK_M, dim), lambda i, h=half_blocks: (i + h, 0)),
            pl.BlockSpec((_NUM_EXPERTS, dim), lambda i: (0, 0)),
        ],
        out_specs=[
            pl.BlockSpec((_BLOCK_M, 2), lambda i: (i, 0)),
            pl.BlockSpec((_BLOCK_M, 2), lambda i: (i, 0)),
        ],
        out_shape=[
            jax.ShapeDtypeStruct((n_tokens // 2, 2), jnp.float32),
            jax.ShapeDtypeStruct((n_tokens // 2, 2), jnp.float32),
        ],
        compiler_params=pltpu.CompilerParams(
            dimension_semantics=("arbitrary",),
            vmem_limit_bytes=64 * 1024 * 1024,
        ),
        interpret=interpret,
    )(x, x, W_g)
    return jnp.concatenate([out_a, out_b], axis=0)
